# baseline (speedup 1.0000x reference)
import functools

import jax
import jax.numpy as jnp
from jax import lax
from jax.experimental import pallas as pl
from jax.experimental.pallas import tpu as pltpu
from jax.experimental.pallas import tpu_sc as plsc

N = 10000
E = 320000
DIM_IN = 128
DIM_H = 16
HEADS = 8
DIM_OUT = 64

NPAD = 10240
JUNK = N
NW = 32
B = 128
EPW = 10496
EPAD = NW * EPW
ROWS_PER_TILE = NPAD // 16


D_SC = 64
CW = D_SC + 16
NV = D_SC // 16


NC = EPW // B
MSG_BYTES = B * CW * 4


def _edge_kernel_fn(H, NG):
    CH = D_SC // H

    def body(*args):
        (xls, xrs, rest) = (args[:NG], args[NG:2 * NG], args[2 * NG:])
        (sd_hbm, att_hbm, out_hbm,
         sd0, sd1, ds0, ds1, xl0, xl1, xr0, xr1, m0, m1, att_s,
         sI0, sI1, sG0, sG1, sS0, sS1, accum) = rest
        sd_v, ds_v = (sd0, sd1), (ds0, ds1)
        xl_v, xr_v, msg_v = (xl0, xl1), (xr0, xr1), (m0, m1)
        semI, semG, semS = (sI0, sI1), (sG0, sG1), (sS0, sS1)
        c = lax.axis_index("c")
        s = lax.axis_index("s")
        wid = s * 2 + c
        wbase = wid * EPW

        pltpu.sync_copy(att_hbm, att_s)
        iota16 = lax.broadcasted_iota(jnp.int32, (16,), 0)
        zero16 = jnp.zeros((16,), jnp.float32)

        def zrow(i, carry):
            for j in range(CW // 16):
                m0[i, pl.ds(16 * j, 16)] = zero16
                m1[i, pl.ds(16 * j, 16)] = zero16
            return carry
        lax.fori_loop(0, B, zrow, 0)

        def wait_idx(b):
            pltpu.make_async_copy(
                sd_hbm.at[:, pl.ds(0, B)], sd_v[b], semI[b]).wait()

        def wait_gathers(b, g):
            pltpu.make_async_copy(
                xls[g].at[pl.ds(0, B)], xl_v[b], semG[b]).wait()
            pltpu.make_async_copy(
                xrs[g].at[pl.ds(0, B)], xr_v[b], semG[b]).wait()

        def wait_scatter(b):
            pltpu.make_async_copy(
                msg_v[b], accum.at[ds_v[b]], semS[b]).wait()

        def issue_idx(b, ci):
            pltpu.async_copy(
                sd_hbm.at[:, pl.ds(wbase + ci * B, B)], sd_v[b], semI[b])

        def issue_gathers(b, g):
            pltpu.async_copy(xls[g].at[sd_v[b].at[0]], xl_v[b], semG[b])
            pltpu.async_copy(xrs[g].at[sd_v[b].at[1]], xr_v[b], semG[b])

        def compute_chunk(b, g):
            def tblock(t, carry):
                evec = iota16 + t * 16
                for h in range(H):
                    alpha = zero16
                    for cc in range(CH):
                        col = h * CH + cc
                        cvec = jnp.full((16,), col, jnp.int32)
                        a = plsc.load_gather(xl_v[b], [evec, cvec])
                        bb = plsc.load_gather(xr_v[b], [evec, cvec])
                        sv = a + bb
                        tv = jnp.maximum(sv, 0.2 * sv)
                        att_sp = plsc.load_gather(
                            att_s, [jnp.full((16,), g * D_SC + col, jnp.int32)])
                        alpha = alpha + tv * att_sp
                    exv = jnp.exp(alpha)
                    for cc in range(CH):
                        col = h * CH + cc
                        cvec = jnp.full((16,), col, jnp.int32)
                        a = plsc.load_gather(xl_v[b], [evec, cvec])
                        plsc.store_scatter(msg_v[b], [evec, cvec], a * exv)
                    plsc.store_scatter(
                        msg_v[b], [evec, jnp.full((16,), D_SC + h, jnp.int32)],
                        exv)
                return carry
            lax.fori_loop(0, B // 16, tblock, 0)

        for g in range(NG):
            for k in range(ROWS_PER_TILE // B):
                pltpu.sync_copy(
                    m0, accum.at[pl.ds(s * ROWS_PER_TILE + k * B, B)])
            plsc.subcore_barrier()

            junk16 = jnp.full((16,), JUNK, jnp.int32)
            def jrow(q, carry):
                ds_v[0][pl.ds(16 * q, 16)] = junk16
                ds_v[1][pl.ds(16 * q, 16)] = junk16
                return carry
            lax.fori_loop(0, B // 16, jrow, 0)
            pltpu.async_copy(msg_v[0], accum.at[ds_v[0]], semS[0], add=True)
            pltpu.async_copy(msg_v[1], accum.at[ds_v[1]], semS[1], add=True)
            cpI0 = pltpu.async_copy(
                sd_hbm.at[:, pl.ds(wbase, B)], sd_v[0], semI[0])
            pltpu.async_copy(
                sd_hbm.at[:, pl.ds(wbase + B, B)], sd_v[1], semI[1])
            cpI0.wait()
            issue_gathers(0, g)

            def pair_body(ci2, carry):
                for b in (0, 1):
                    ci = 2 * ci2 + b
                    wait_idx(1 - b)
                    issue_gathers(1 - b, g)
                    wait_gathers(b, g)
                    wait_scatter(b)
                    for q in range(B // 16):
                        ds_v[b][pl.ds(16 * q, 16)] = sd_v[b][1, pl.ds(16 * q, 16)]
                    issue_idx(b, ci + 2)
                    compute_chunk(b, g)
                    pltpu.async_copy(
                        msg_v[b], accum.at[ds_v[b]], semS[b], add=True)
                return carry

            lax.fori_loop(0, NC // 2, pair_body, 0)

            wait_gathers(0, g)
            wait_idx(1)
            wait_scatter(0)
            wait_scatter(1)
            plsc.subcore_barrier()

            pltpu.sync_copy(
                accum.at[pl.ds(s * ROWS_PER_TILE, ROWS_PER_TILE)],
                out_hbm.at[g, c, pl.ds(s * ROWS_PER_TILE, ROWS_PER_TILE)])
            plsc.subcore_barrier()

    return body


def _make_edge_call(H, NG):
    body = _edge_kernel_fn(H, NG)
    return pl.kernel(
        body,
        out_type=jax.ShapeDtypeStruct((NG, 2, NPAD, CW), jnp.float32),
        mesh=plsc.VectorSubcoreMesh(core_axis_name="c", subcore_axis_name="s"),
        compiler_params=pltpu.CompilerParams(
            needs_layout_passes=False, use_tc_tiling_on_sc=False),
        scratch_types=[
            pltpu.VMEM((2, B), jnp.int32),
            pltpu.VMEM((2, B), jnp.int32),
            pltpu.VMEM((B,), jnp.int32),
            pltpu.VMEM((B,), jnp.int32),
            pltpu.VMEM((B, D_SC), jnp.float32),
            pltpu.VMEM((B, D_SC), jnp.float32),
            pltpu.VMEM((B, D_SC), jnp.float32),
            pltpu.VMEM((B, D_SC), jnp.float32),
            pltpu.VMEM((B, CW), jnp.float32),
            pltpu.VMEM((B, CW), jnp.float32),
            pltpu.VMEM((NG * D_SC,), jnp.float32),
            pltpu.SemaphoreType.DMA,
            pltpu.SemaphoreType.DMA,
            pltpu.SemaphoreType.DMA,
            pltpu.SemaphoreType.DMA,
            pltpu.SemaphoreType.DMA,
            pltpu.SemaphoreType.DMA,
            pltpu.VMEM_SHARED((NPAD, CW), jnp.float32),
        ],
    )


RB = 1024


def _mm1_body(x_ref, w_ref, b_ref, xla_ref, xlb_ref, xra_ref, xrb_ref):
    acc = jnp.dot(x_ref[...], w_ref[...],
                  preferred_element_type=jnp.float32) + b_ref[...]
    xla_ref[...] = acc[:, 0:64]
    xlb_ref[...] = acc[:, 64:128]
    xra_ref[...] = acc[:, 128:192]
    xrb_ref[...] = acc[:, 192:256]


def _mm1_call(x_pad, wcat, bcat):
    grid = (NPAD // RB,)
    tbl = jax.ShapeDtypeStruct((NPAD, 64), jnp.float32)
    return pl.pallas_call(
        _mm1_body,
        grid=grid,
        in_specs=[
            pl.BlockSpec((RB, DIM_IN), lambda i: (i, 0)),
            pl.BlockSpec((DIM_IN, 2 * DIM_IN), lambda i: (0, 0)),
            pl.BlockSpec((1, 2 * DIM_IN), lambda i: (0, 0)),
        ],
        out_specs=[pl.BlockSpec((RB, 64), lambda i: (i, 0))] * 4,
        out_shape=[tbl] * 4,
    )(x_pad, wcat, bcat)


def _mid_body(acc_ref, e4_ref, b1_ref, w2_ref, b2_ref, h2l_ref, h2r_ref):
    hs = []
    for g in range(2):
        a = acc_ref[g, 0] + acc_ref[g, 1]
        num = a[:, :D_SC]
        den = a[:, D_SC:D_SC + 4]
        r = 1.0 / (den + 1e-16)
        r64 = jnp.dot(r, e4_ref[...], preferred_element_type=jnp.float32)
        hs.append(num * r64)
    h = jnp.concatenate(hs, axis=1) + b1_ref[...]
    h = jnp.where(h > 0, h, jnp.exp(jnp.minimum(h, 0.0)) - 1.0)
    h2 = jnp.dot(h, w2_ref[...], preferred_element_type=jnp.float32) + b2_ref[...]
    h2l_ref[...] = h2[:, :DIM_OUT]
    h2r_ref[...] = h2[:, DIM_OUT:]


def _mid_call(accum1, e4, b1, w2cat, b2cat):
    grid = (NPAD // RB,)
    return pl.pallas_call(
        _mid_body,
        grid=grid,
        in_specs=[
            pl.BlockSpec((2, 2, RB, CW), lambda i: (0, 0, i, 0)),
            pl.BlockSpec((4, D_SC), lambda i: (0, 0)),
            pl.BlockSpec((1, DIM_IN), lambda i: (0, 0)),
            pl.BlockSpec((DIM_IN, 2 * DIM_OUT), lambda i: (0, 0)),
            pl.BlockSpec((1, 2 * DIM_OUT), lambda i: (0, 0)),
        ],
        out_specs=[
            pl.BlockSpec((RB, DIM_OUT), lambda i: (i, 0)),
            pl.BlockSpec((RB, DIM_OUT), lambda i: (i, 0)),
        ],
        out_shape=[
            jax.ShapeDtypeStruct((NPAD, DIM_OUT), jnp.float32),
            jax.ShapeDtypeStruct((NPAD, DIM_OUT), jnp.float32),
        ],
    )(accum1, e4, b1, w2cat, b2cat)


def _final_body(acc_ref, b2_ref, out_ref):
    a = acc_ref[0, 0] + acc_ref[0, 1]
    num = a[:, :DIM_OUT]
    den = a[:, DIM_OUT:DIM_OUT + 1]
    o = num / (den + 1e-16) + b2_ref[...]
    m = jnp.max(o, axis=1, keepdims=True)
    ls = m + jnp.log(jnp.sum(jnp.exp(o - m), axis=1, keepdims=True))
    out_ref[...] = o - ls


def _final_call(accum2, bias2):
    grid = (NPAD // RB,)
    return pl.pallas_call(
        _final_body,
        grid=grid,
        in_specs=[
            pl.BlockSpec((1, 2, RB, CW), lambda i: (0, 0, i, 0)),
            pl.BlockSpec((1, DIM_OUT), lambda i: (0, 0)),
        ],
        out_specs=pl.BlockSpec((RB, DIM_OUT), lambda i: (i, 0)),
        out_shape=jax.ShapeDtypeStruct((NPAD, DIM_OUT), jnp.float32),
    )(accum2, bias2)


def kernel(x, edge_index, Wl1, bl1, Wr1, br1, att1, bias1,
           Wl2, bl2, Wr2, br2, att2, bias2):
    f32 = jnp.float32
    src0 = edge_index[0]
    dst0 = edge_index[1]
    dstm = jnp.where(src0 != dst0, dst0, jnp.int32(JUNK))
    loops = jnp.arange(N, dtype=jnp.int32)
    npad_e = EPAD + 2 * B - (E + N)
    src = jnp.concatenate([src0, loops, jnp.zeros((npad_e,), jnp.int32)])
    dst = jnp.concatenate([dstm, loops, jnp.full((npad_e,), JUNK, jnp.int32)])
    sd = jnp.stack([src, dst])

    x_pad = jnp.pad(x, ((0, NPAD - N), (0, 0)))
    wcat1 = jnp.concatenate([Wl1, Wr1], axis=1)
    bcat1 = jnp.concatenate([bl1, br1]).reshape(1, -1)
    w2cat = jnp.concatenate([Wl2, Wr2], axis=1)
    b2cat = jnp.concatenate([bl2, br2]).reshape(1, -1)
    att1r = att1.reshape(-1)
    att2r = att2.reshape(-1)
    e4 = jnp.repeat(jnp.eye(4, dtype=f32), DIM_H, axis=1)

    xla, xlb, xra, xrb = _mm1_call(x_pad, wcat1, bcat1)
    accum1 = _make_edge_call(4, 2)(xla, xlb, xra, xrb, sd, att1r)
    h2l, h2r = _mid_call(accum1, e4, bias1.reshape(1, -1), w2cat, b2cat)

    accum2 = _make_edge_call(1, 1)(h2l, h2r, sd, att2r)
    out = _final_call(accum2, bias2.reshape(1, -1))
    return out[:N]


# speedup vs baseline: 1.4435x; 1.4435x over previous
import functools

import jax
import jax.numpy as jnp
from jax import lax
from jax.experimental import pallas as pl
from jax.experimental.pallas import tpu as pltpu
from jax.experimental.pallas import tpu_sc as plsc

N = 10000
E = 320000
DIM_IN = 128
DIM_H = 16
HEADS = 8
DIM_OUT = 64

NPAD = 10240
JUNK = N
NW = 32
B = 128
EPW = 10496
EPAD = NW * EPW
ROWS_PER_TILE = NPAD // 16


D_SC = 64
CW = D_SC + 16
NV = D_SC // 16


NC = EPW // B
MSG_BYTES = B * CW * 4


def _edge_kernel_fn(H, NG):
    CH = D_SC // H

    def body(*args):
        (xls, xrs, rest) = (args[:NG], args[NG:2 * NG], args[2 * NG:])
        (sd_hbm, att_hbm, out_hbm,
         sd0, sd1, ds0, ds1, xl0, xl1, xr0, xr1, m0, m1, att_s,
         sI0, sI1, sG0, sG1, sS0, sS1, accum) = rest
        sd_v, ds_v = (sd0, sd1), (ds0, ds1)
        xl_v, xr_v, msg_v = (xl0, xl1), (xr0, xr1), (m0, m1)
        semI, semG, semS = (sI0, sI1), (sG0, sG1), (sS0, sS1)
        c = lax.axis_index("c")
        s = lax.axis_index("s")
        wid = s * 2 + c
        wbase = wid * EPW

        pltpu.sync_copy(att_hbm, att_s)
        iota16 = lax.broadcasted_iota(jnp.int32, (16,), 0)
        zero16 = jnp.zeros((16,), jnp.float32)

        def zrow(i, carry):
            for j in range(CW // 16):
                m0[i, pl.ds(16 * j, 16)] = zero16
                m1[i, pl.ds(16 * j, 16)] = zero16
            return carry
        lax.fori_loop(0, B, zrow, 0)

        def wait_idx(b):
            pltpu.make_async_copy(
                sd_hbm.at[:, pl.ds(0, B)], sd_v[b], semI[b]).wait()

        def wait_gathers(b, g):
            pltpu.make_async_copy(
                xls[g].at[pl.ds(0, B)], xl_v[b], semG[b]).wait()
            pltpu.make_async_copy(
                xrs[g].at[pl.ds(0, B)], xr_v[b], semG[b]).wait()

        def wait_scatter(b):
            pltpu.make_async_copy(
                msg_v[b], accum.at[ds_v[b]], semS[b]).wait()

        def issue_idx(b, ci):
            pltpu.async_copy(
                sd_hbm.at[:, pl.ds(wbase + ci * B, B)], sd_v[b], semI[b])

        def issue_gathers(b, g):
            pltpu.async_copy(xls[g].at[sd_v[b].at[0]], xl_v[b], semG[b])
            pltpu.async_copy(xrs[g].at[sd_v[b].at[1]], xr_v[b], semG[b])

        VPH = NV // H
        perms = [iota16 ^ st for st in (8, 4, 2, 1)]

        def compute_chunk(b, g):
            def erow(i, carry):
                den = zero16
                for h in range(H):
                    acc = zero16
                    avs = []
                    for k in range(VPH):
                        j = h * VPH + k
                        a = xl_v[b][i, pl.ds(16 * j, 16)]
                        bb = xr_v[b][i, pl.ds(16 * j, 16)]
                        avs.append(a)
                        sv = a + bb
                        tv = jnp.maximum(sv, 0.2 * sv)
                        acc = acc + tv * att_s[g * NV + j]
                    for p in perms:
                        acc = acc + acc.at[p].get(mode="promise_in_bounds")
                    exv = jnp.exp(acc)
                    for k in range(VPH):
                        j = h * VPH + k
                        msg_v[b][i, pl.ds(16 * j, 16)] = avs[k] * exv
                    den = jnp.where(iota16 == h, exv, den)
                msg_v[b][i, pl.ds(D_SC, 16)] = den
                return carry
            lax.fori_loop(0, B, erow, 0)

        for g in range(NG):
            for k in range(ROWS_PER_TILE // B):
                pltpu.sync_copy(
                    m0, accum.at[pl.ds(s * ROWS_PER_TILE + k * B, B)])
            plsc.subcore_barrier()

            junk16 = jnp.full((16,), JUNK, jnp.int32)
            def jrow(q, carry):
                ds_v[0][pl.ds(16 * q, 16)] = junk16
                ds_v[1][pl.ds(16 * q, 16)] = junk16
                return carry
            lax.fori_loop(0, B // 16, jrow, 0)
            pltpu.async_copy(msg_v[0], accum.at[ds_v[0]], semS[0], add=True)
            pltpu.async_copy(msg_v[1], accum.at[ds_v[1]], semS[1], add=True)
            cpI0 = pltpu.async_copy(
                sd_hbm.at[:, pl.ds(wbase, B)], sd_v[0], semI[0])
            pltpu.async_copy(
                sd_hbm.at[:, pl.ds(wbase + B, B)], sd_v[1], semI[1])
            cpI0.wait()
            issue_gathers(0, g)

            def pair_body(ci2, carry):
                for b in (0, 1):
                    ci = 2 * ci2 + b
                    wait_idx(1 - b)
                    issue_gathers(1 - b, g)
                    wait_gathers(b, g)
                    wait_scatter(b)
                    for q in range(B // 16):
                        ds_v[b][pl.ds(16 * q, 16)] = sd_v[b][1, pl.ds(16 * q, 16)]
                    issue_idx(b, ci + 2)
                    compute_chunk(b, g)
                    pltpu.async_copy(
                        msg_v[b], accum.at[ds_v[b]], semS[b], add=True)
                return carry

            lax.fori_loop(0, NC // 2, pair_body, 0)

            wait_gathers(0, g)
            wait_idx(1)
            wait_scatter(0)
            wait_scatter(1)
            plsc.subcore_barrier()

            pltpu.sync_copy(
                accum.at[pl.ds(s * ROWS_PER_TILE, ROWS_PER_TILE)],
                out_hbm.at[g, c, pl.ds(s * ROWS_PER_TILE, ROWS_PER_TILE)])
            plsc.subcore_barrier()

    return body


def _make_edge_call(H, NG):
    body = _edge_kernel_fn(H, NG)
    return pl.kernel(
        body,
        out_type=jax.ShapeDtypeStruct((NG, 2, NPAD, CW), jnp.float32),
        mesh=plsc.VectorSubcoreMesh(core_axis_name="c", subcore_axis_name="s"),
        compiler_params=pltpu.CompilerParams(
            needs_layout_passes=False, use_tc_tiling_on_sc=False),
        scratch_types=[
            pltpu.VMEM((2, B), jnp.int32),
            pltpu.VMEM((2, B), jnp.int32),
            pltpu.VMEM((B,), jnp.int32),
            pltpu.VMEM((B,), jnp.int32),
            pltpu.VMEM((B, D_SC), jnp.float32),
            pltpu.VMEM((B, D_SC), jnp.float32),
            pltpu.VMEM((B, D_SC), jnp.float32),
            pltpu.VMEM((B, D_SC), jnp.float32),
            pltpu.VMEM((B, CW), jnp.float32),
            pltpu.VMEM((B, CW), jnp.float32),
            pltpu.VMEM((NG * NV, 16), jnp.float32),
            pltpu.SemaphoreType.DMA,
            pltpu.SemaphoreType.DMA,
            pltpu.SemaphoreType.DMA,
            pltpu.SemaphoreType.DMA,
            pltpu.SemaphoreType.DMA,
            pltpu.SemaphoreType.DMA,
            pltpu.VMEM_SHARED((NPAD, CW), jnp.float32),
        ],
    )


RB = 1024


def _mm1_body(x_ref, w_ref, b_ref, xla_ref, xlb_ref, xra_ref, xrb_ref):
    acc = jnp.dot(x_ref[...], w_ref[...],
                  preferred_element_type=jnp.float32) + b_ref[...]
    xla_ref[...] = acc[:, 0:64]
    xlb_ref[...] = acc[:, 64:128]
    xra_ref[...] = acc[:, 128:192]
    xrb_ref[...] = acc[:, 192:256]


def _mm1_call(x_pad, wcat, bcat):
    grid = (NPAD // RB,)
    tbl = jax.ShapeDtypeStruct((NPAD, 64), jnp.float32)
    return pl.pallas_call(
        _mm1_body,
        grid=grid,
        in_specs=[
            pl.BlockSpec((RB, DIM_IN), lambda i: (i, 0)),
            pl.BlockSpec((DIM_IN, 2 * DIM_IN), lambda i: (0, 0)),
            pl.BlockSpec((1, 2 * DIM_IN), lambda i: (0, 0)),
        ],
        out_specs=[pl.BlockSpec((RB, 64), lambda i: (i, 0))] * 4,
        out_shape=[tbl] * 4,
    )(x_pad, wcat, bcat)


def _mid_body(acc_ref, e4_ref, b1_ref, w2_ref, b2_ref, h2l_ref, h2r_ref):
    hs = []
    for g in range(2):
        a = acc_ref[g, 0] + acc_ref[g, 1]
        num = a[:, :D_SC]
        den = a[:, D_SC:D_SC + 4]
        r = 1.0 / (den + 1e-16)
        r64 = jnp.dot(r, e4_ref[...], preferred_element_type=jnp.float32)
        hs.append(num * r64)
    h = jnp.concatenate(hs, axis=1) + b1_ref[...]
    h = jnp.where(h > 0, h, jnp.exp(jnp.minimum(h, 0.0)) - 1.0)
    h2 = jnp.dot(h, w2_ref[...], preferred_element_type=jnp.float32) + b2_ref[...]
    h2l_ref[...] = h2[:, :DIM_OUT]
    h2r_ref[...] = h2[:, DIM_OUT:]


def _mid_call(accum1, e4, b1, w2cat, b2cat):
    grid = (NPAD // RB,)
    return pl.pallas_call(
        _mid_body,
        grid=grid,
        in_specs=[
            pl.BlockSpec((2, 2, RB, CW), lambda i: (0, 0, i, 0)),
            pl.BlockSpec((4, D_SC), lambda i: (0, 0)),
            pl.BlockSpec((1, DIM_IN), lambda i: (0, 0)),
            pl.BlockSpec((DIM_IN, 2 * DIM_OUT), lambda i: (0, 0)),
            pl.BlockSpec((1, 2 * DIM_OUT), lambda i: (0, 0)),
        ],
        out_specs=[
            pl.BlockSpec((RB, DIM_OUT), lambda i: (i, 0)),
            pl.BlockSpec((RB, DIM_OUT), lambda i: (i, 0)),
        ],
        out_shape=[
            jax.ShapeDtypeStruct((NPAD, DIM_OUT), jnp.float32),
            jax.ShapeDtypeStruct((NPAD, DIM_OUT), jnp.float32),
        ],
    )(accum1, e4, b1, w2cat, b2cat)


def _final_body(acc_ref, b2_ref, out_ref):
    a = acc_ref[0, 0] + acc_ref[0, 1]
    num = a[:, :DIM_OUT]
    den = a[:, DIM_OUT:DIM_OUT + 1]
    o = num / (den + 1e-16) + b2_ref[...]
    m = jnp.max(o, axis=1, keepdims=True)
    ls = m + jnp.log(jnp.sum(jnp.exp(o - m), axis=1, keepdims=True))
    out_ref[...] = o - ls


def _final_call(accum2, bias2):
    grid = (NPAD // RB,)
    return pl.pallas_call(
        _final_body,
        grid=grid,
        in_specs=[
            pl.BlockSpec((1, 2, RB, CW), lambda i: (0, 0, i, 0)),
            pl.BlockSpec((1, DIM_OUT), lambda i: (0, 0)),
        ],
        out_specs=pl.BlockSpec((RB, DIM_OUT), lambda i: (i, 0)),
        out_shape=jax.ShapeDtypeStruct((NPAD, DIM_OUT), jnp.float32),
    )(accum2, bias2)


def kernel(x, edge_index, Wl1, bl1, Wr1, br1, att1, bias1,
           Wl2, bl2, Wr2, br2, att2, bias2):
    f32 = jnp.float32
    src0 = edge_index[0]
    dst0 = edge_index[1]
    dstm = jnp.where(src0 != dst0, dst0, jnp.int32(JUNK))
    loops = jnp.arange(N, dtype=jnp.int32)
    npad_e = EPAD + 2 * B - (E + N)
    src = jnp.concatenate([src0, loops, jnp.zeros((npad_e,), jnp.int32)])
    dst = jnp.concatenate([dstm, loops, jnp.full((npad_e,), JUNK, jnp.int32)])
    sd = jnp.stack([src, dst])

    x_pad = jnp.pad(x, ((0, NPAD - N), (0, 0)))
    wcat1 = jnp.concatenate([Wl1, Wr1], axis=1)
    bcat1 = jnp.concatenate([bl1, br1]).reshape(1, -1)
    w2cat = jnp.concatenate([Wl2, Wr2], axis=1)
    b2cat = jnp.concatenate([bl2, br2]).reshape(1, -1)
    att1r = att1.reshape(HEADS, DIM_H)
    att2r = att2.reshape(DIM_OUT // 16, 16)
    e4 = jnp.repeat(jnp.eye(4, dtype=f32), DIM_H, axis=1)

    xla, xlb, xra, xrb = _mm1_call(x_pad, wcat1, bcat1)
    accum1 = _make_edge_call(4, 2)(xla, xlb, xra, xrb, sd, att1r)
    h2l, h2r = _mid_call(accum1, e4, bias1.reshape(1, -1), w2cat, b2cat)

    accum2 = _make_edge_call(1, 1)(h2l, h2r, sd, att2r)
    out = _final_call(accum2, bias2.reshape(1, -1))
    return out[:N]
